# baseline (speedup 1.0000x reference)
import functools

import jax
import jax.numpy as jnp
from jax.experimental import pallas as pl
from jax.experimental.pallas import tpu as pltpu

_VMEM_LIMIT = 100 * 1024 * 1024


def _mm_kernel(x_ref, w_ref, b_ref, o_ref):
    acc = jnp.dot(x_ref[...], w_ref[...], preferred_element_type=jnp.float32)
    o_ref[...] = (acc + b_ref[...]).astype(o_ref.dtype)


@functools.partial(jax.jit, static_argnames=("tile",))
def _grappa(x, w, b2, tile):
    B, F_in = x.shape
    F_out = w.shape[1]
    grid = (pl.cdiv(B, tile),)
    return pl.pallas_call(
        _mm_kernel,
        out_shape=jax.ShapeDtypeStruct((B, F_out), x.dtype),
        grid=grid,
        in_specs=[
            pl.BlockSpec((tile, F_in), lambda i: (i, 0)),
            pl.BlockSpec((F_in, F_out), lambda i: (0, 0)),
            pl.BlockSpec((1, F_out), lambda i: (0, 0)),
        ],
        out_specs=pl.BlockSpec((tile, F_out), lambda i: (i, 0)),
        compiler_params=pltpu.CompilerParams(
            dimension_semantics=("arbitrary",) if grid[0] == 1 else ("parallel",),
            vmem_limit_bytes=_VMEM_LIMIT,
        ),
        cost_estimate=pl.CostEstimate(
            flops=2 * B * F_in * F_out,
            transcendentals=0,
            bytes_accessed=(B * F_in + F_in * F_out + B * F_out) * 4,
        ),
    )(x, w, b2)


def _pick_tile(B: int) -> int:
    for tile in (16384, 8192, 4096, 1024, 512, 256, 128, 8):
        if B % tile == 0 and B // tile >= 2:
            return tile
    return B


def kernel(x, w, b):
    F_out = w.shape[1]
    return _grappa(x, w, b.reshape(1, F_out).astype(jnp.float32), _pick_tile(x.shape[0]))
